# baseline (speedup 1.0000x reference)
import functools
import jax
import jax.numpy as jnp
from jax import lax
from jax.experimental import pallas as pl
from jax.experimental.pallas import tpu as pltpu
from jax.experimental.pallas import tpu_sc as plsc

NC = 2
NS = 16
NW = NC * NS

B, C, H, W = 16, 192, 56, 56
G = 4
SITES = B * H * W
SB = 8
BLOCKS = SITES // SB
BPW = BLOCKS // NW
NB = 7
NCHUNK = BPW // NB
LANES = 16
OB = C // LANES


def _compute_chunk(colt_v, rows_v, out_v):
    subs = [jnp.full((LANES,), su, dtype=jnp.int32) for su in range(SB)]

    @plsc.parallel_loop(0, OB)
    def oloop(o):
        cols = [colt_v[G * o + j, :] for j in range(G)]
        oc = o * LANES
        for blk in range(NB):
            r8 = rows_v.at[blk]
            for half in range(SB // 4):
                gs = []
                for su4 in range(4):
                    su = half * 4 + su4
                    gs.append([
                        plsc.load_gather(r8, [subs[su], cols[j]])
                        for j in range(G)
                    ])
                for su4 in range(4):
                    su = half * 4 + su4
                    g0, g1, g2, g3 = gs[su4]
                    out_v[blk, su, pl.ds(oc, LANES)] = jnp.maximum(
                        jnp.maximum(g0, g1), jnp.maximum(g2, g3)
                    )


def _body(
    x_hbm, colt_hbm, out_hbm,
    colt_v, rows_v0, rows_v1, out_v0, out_v1,
    isem0, isem1, osem0, osem1,
):
    c = lax.axis_index("c")
    s = lax.axis_index("s")
    w = s * NC + c
    base = w * BPW
    pltpu.sync_copy(colt_hbm, colt_v)

    ibufs = (rows_v0, rows_v1)
    isems = (isem0, isem1)
    obufs = (out_v0, out_v1)
    osems = (osem0, osem1)

    def load_start(ci, buf, sem):
        pltpu.async_copy(x_hbm.at[pl.ds(base + ci * NB, NB)], buf, sem)

    def load_wait(buf, sem):
        pltpu.make_async_copy(x_hbm.at[pl.ds(base, NB)], buf, sem).wait()

    def store_start(ci, buf, sem):
        pltpu.async_copy(buf, out_hbm.at[pl.ds(base + ci * NB, NB)], sem)

    def store_wait(buf, sem):
        pltpu.make_async_copy(buf, out_hbm.at[pl.ds(base, NB)], sem).wait()

    load_start(0, rows_v0, isem0)
    load_start(1, rows_v1, isem1)

    def pair(g, carry):
        for bu in range(2):
            ci = g * 2 + bu
            load_wait(ibufs[bu], isems[bu])

            @pl.when(ci >= 2)
            def _():
                store_wait(obufs[bu], osems[bu])

            _compute_chunk(colt_v, ibufs[bu], obufs[bu])
            store_start(ci, obufs[bu], osems[bu])

            @pl.when(ci + 2 < NCHUNK)
            def _():
                load_start(ci + 2, ibufs[bu], isems[bu])

        return carry

    lax.fori_loop(0, NCHUNK // 2, pair, 0)
    store_wait(out_v0, osem0)
    store_wait(out_v1, osem1)


@jax.jit
def _run(x3, colt):
    mesh = plsc.VectorSubcoreMesh(core_axis_name="c", subcore_axis_name="s")
    f = functools.partial(
        pl.kernel,
        out_type=jax.ShapeDtypeStruct((BLOCKS, SB, C), jnp.float32),
        mesh=mesh,
        compiler_params=pltpu.CompilerParams(
            use_tc_tiling_on_sc=True, needs_layout_passes=False
        ),
        scratch_types=[
            pltpu.VMEM((G * OB, LANES), jnp.int32),
            pltpu.VMEM((NB, SB, C), jnp.float32),
            pltpu.VMEM((NB, SB, C), jnp.float32),
            pltpu.VMEM((NB, SB, C), jnp.float32),
            pltpu.VMEM((NB, SB, C), jnp.float32),
            pltpu.SemaphoreType.DMA,
            pltpu.SemaphoreType.DMA,
            pltpu.SemaphoreType.DMA,
            pltpu.SemaphoreType.DMA,
        ],
    )(_body)
    return f(x3, colt)


def kernel(x, index):
    idx4 = (index.astype(jnp.int32) % C).reshape(C, G)
    colt = idx4.reshape(OB, LANES, G).transpose(0, 2, 1).reshape(G * OB, LANES)
    x3 = jnp.transpose(x, (0, 2, 3, 1)).reshape(BLOCKS, SB, C)
    o3 = _run(x3, colt)
    return o3.reshape(B, H, W, C).transpose(0, 3, 1, 2)


# speedup vs baseline: 1.0639x; 1.0639x over previous
import functools
import jax
import jax.numpy as jnp
from jax import lax
from jax.experimental import pallas as pl
from jax.experimental.pallas import tpu as pltpu
from jax.experimental.pallas import tpu_sc as plsc

NC = 2
NS = 16
NW = NC * NS

B, C, H, W = 16, 192, 56, 56
G = 4
SITES = B * H * W
SB = 8
BLOCKS = SITES // SB
BPW = BLOCKS // NW
NB = 7
NCHUNK = BPW // NB
LANES = 16
OB = C // LANES


def _compute_chunk(colt_v, rows_v, out_v):
    for o in range(OB):
        cols = [colt_v[G * o + j, :] for j in range(G)]
        subs = [jnp.full((LANES,), su, dtype=jnp.int32) for su in range(SB)]

        @plsc.parallel_loop(0, NB)
        def bloop(blk, cols=cols, subs=subs, o=o):
            r8 = rows_v.at[blk]
            for half in range(SB // 4):
                gs = []
                for su4 in range(4):
                    su = half * 4 + su4
                    gs.append([
                        plsc.load_gather(r8, [subs[su], cols[j]])
                        for j in range(G)
                    ])
                for su4 in range(4):
                    su = half * 4 + su4
                    g0, g1, g2, g3 = gs[su4]
                    out_v[blk, su, pl.ds(o * LANES, LANES)] = jnp.maximum(
                        jnp.maximum(g0, g1), jnp.maximum(g2, g3)
                    )


def _body(
    x_hbm, colt_hbm, out_hbm,
    colt_v, rows_v0, rows_v1, out_v0, out_v1,
    isem0, isem1, osem0, osem1,
):
    c = lax.axis_index("c")
    s = lax.axis_index("s")
    w = s * NC + c
    base = w * BPW
    pltpu.sync_copy(colt_hbm, colt_v)

    ibufs = (rows_v0, rows_v1)
    isems = (isem0, isem1)
    obufs = (out_v0, out_v1)
    osems = (osem0, osem1)

    def load_start(ci, buf, sem):
        pltpu.async_copy(x_hbm.at[pl.ds(base + ci * NB, NB)], buf, sem)

    def load_wait(buf, sem):
        pltpu.make_async_copy(x_hbm.at[pl.ds(base, NB)], buf, sem).wait()

    def store_start(ci, buf, sem):
        pltpu.async_copy(buf, out_hbm.at[pl.ds(base + ci * NB, NB)], sem)

    def store_wait(buf, sem):
        pltpu.make_async_copy(buf, out_hbm.at[pl.ds(base, NB)], sem).wait()

    load_start(0, rows_v0, isem0)
    load_start(1, rows_v1, isem1)

    def pair(g, carry):
        for bu in range(2):
            ci = g * 2 + bu
            load_wait(ibufs[bu], isems[bu])

            @pl.when(ci >= 2)
            def _():
                store_wait(obufs[bu], osems[bu])

            _compute_chunk(colt_v, ibufs[bu], obufs[bu])
            store_start(ci, obufs[bu], osems[bu])

            @pl.when(ci + 2 < NCHUNK)
            def _():
                load_start(ci + 2, ibufs[bu], isems[bu])

        return carry

    lax.fori_loop(0, NCHUNK // 2, pair, 0)
    store_wait(out_v0, osem0)
    store_wait(out_v1, osem1)


@jax.jit
def _run(x3, colt):
    mesh = plsc.VectorSubcoreMesh(core_axis_name="c", subcore_axis_name="s")
    f = functools.partial(
        pl.kernel,
        out_type=jax.ShapeDtypeStruct((BLOCKS, SB, C), jnp.float32),
        mesh=mesh,
        compiler_params=pltpu.CompilerParams(
            use_tc_tiling_on_sc=True, needs_layout_passes=False
        ),
        scratch_types=[
            pltpu.VMEM((G * OB, LANES), jnp.int32),
            pltpu.VMEM((NB, SB, C), jnp.float32),
            pltpu.VMEM((NB, SB, C), jnp.float32),
            pltpu.VMEM((NB, SB, C), jnp.float32),
            pltpu.VMEM((NB, SB, C), jnp.float32),
            pltpu.SemaphoreType.DMA,
            pltpu.SemaphoreType.DMA,
            pltpu.SemaphoreType.DMA,
            pltpu.SemaphoreType.DMA,
        ],
    )(_body)
    return f(x3, colt)


def kernel(x, index):
    idx4 = (index.astype(jnp.int32) % C).reshape(C, G)
    colt = idx4.reshape(OB, LANES, G).transpose(0, 2, 1).reshape(G * OB, LANES)
    x3 = jnp.transpose(x, (0, 2, 3, 1)).reshape(BLOCKS, SB, C)
    o3 = _run(x3, colt)
    return o3.reshape(B, H, W, C).transpose(0, 3, 1, 2)
